# baseline (speedup 1.0000x reference)
import functools

import jax
import jax.numpy as jnp
from jax import lax
from jax.experimental import pallas as pl
from jax.experimental.pallas import tpu as pltpu
from jax.experimental.pallas import tpu_sc as plsc

N = 10000
E = 320000
F = 128
NC = 2
NS = 16
NW = NC * NS
NPAD = 10240
EPW = E // NW
K = 128
CH = 80
EPAD = NW * CH * K
HCH = CH // 2
NSL = NPAD // NS
RPW = 624

@functools.cache
def _sc_kernels():
    mesh = plsc.VectorSubcoreMesh(core_axis_name="c", subcore_axis_name="s",
                                  num_cores=NC, num_subcores=NS)
    params = pltpu.CompilerParams(needs_layout_passes=False)
    deg = functools.partial(
        pl.kernel,
        out_type=(jax.ShapeDtypeStruct((NW * NPAD,), jnp.float32),
                  jax.ShapeDtypeStruct((NW * NPAD,), jnp.float32)),
        mesh=mesh,
        compiler_params=params,
        scratch_types=[
            pltpu.VMEM((EPW,), jnp.int32),
            pltpu.VMEM((NPAD,), jnp.float32),
            pltpu.VMEM((NPAD,), jnp.float32),
        ],
    )(_deg_body)
    wk = functools.partial(
        pl.kernel,
        out_type=jax.ShapeDtypeStruct((NW * CH, K), jnp.float32),
        mesh=mesh,
        compiler_params=params,
        scratch_types=[
            pltpu.VMEM((CH, K), jnp.int32),
            pltpu.VMEM((CH, K), jnp.int32),
            pltpu.VMEM((CH, K), jnp.float32),
            pltpu.VMEM((NPAD,), jnp.float32),
            pltpu.VMEM((NPAD,), jnp.float32),
            pltpu.VMEM((NW, NSL), jnp.float32),
            pltpu.VMEM((NSL,), jnp.float32),
            pltpu.VMEM_SHARED((NPAD,), jnp.float32),
            pltpu.VMEM_SHARED((NPAD,), jnp.float32),
        ],
    )(_w_body)
    mp = functools.partial(
        pl.kernel,
        out_type=jax.ShapeDtypeStruct((NC, N, F), jnp.float32),
        mesh=mesh,
        compiler_params=params,
        scratch_types=[
            pltpu.VMEM((HCH, K), jnp.int32),
            pltpu.VMEM((HCH, K), jnp.int32),
            pltpu.VMEM((HCH, K), jnp.float32),
            pltpu.VMEM((K, F), jnp.float32),
            pltpu.VMEM((K, F), jnp.float32),
            pltpu.VMEM_SHARED((N, F), jnp.float32),
            pltpu.SemaphoreType.DMA,
            pltpu.SemaphoreType.DMA,
            pltpu.SemaphoreType.DMA,
            pltpu.SemaphoreType.DMA,
        ],
    )(_mp_body)
    return deg, wk, mp


def _deg_body(src_hbm, dst_hbm, outs_hbm, outd_hbm, idx_v, hist_s, hist_d):
    cid = lax.axis_index("c")
    sid = lax.axis_index("s")
    wid = sid * NC + cid

    def zbody(i, _):
        z = jnp.zeros((16,), jnp.float32)
        hist_s[pl.ds(i * 16, 16)] = z
        hist_d[pl.ds(i * 16, 16)] = z
        return 0

    lax.fori_loop(0, NPAD // 16, zbody, 0)

    ones = jnp.ones((16,), jnp.float32)

    pltpu.sync_copy(src_hbm.at[pl.ds(wid * EPW, EPW)], idx_v)

    def sbody(i, _):
        idx = idx_v[pl.ds(i * 16, 16)]
        plsc.addupdate_scatter(hist_s, [idx], ones)
        return 0

    lax.fori_loop(0, EPW // 16, sbody, 0)

    pltpu.sync_copy(dst_hbm.at[pl.ds(wid * EPW, EPW)], idx_v)

    def dbody(i, _):
        idx = idx_v[pl.ds(i * 16, 16)]
        plsc.addupdate_scatter(hist_d, [idx], ones)
        return 0

    lax.fori_loop(0, EPW // 16, dbody, 0)

    pltpu.sync_copy(hist_s, outs_hbm.at[pl.ds(wid * NPAD, NPAD)])
    pltpu.sync_copy(hist_d, outd_hbm.at[pl.ds(wid * NPAD, NPAD)])


def _rsqrt16(d):
    i = lax.bitcast_convert_type(d, jnp.int32)
    i = jnp.int32(0x5F3759DF) - lax.shift_right_logical(i, 1)
    y = lax.bitcast_convert_type(i, jnp.float32)
    for _ in range(3):
        y = y * (1.5 - 0.5 * d * y * y)
    return jnp.where(d > 0.0, y, 0.0)


def _w_body(ew_hbm, src_hbm, dst_hbm, degs_hbm, degd_hbm, out_hbm,
            src_v, dst_v, w_v, ns_v, nd_v, part_v, slice_v, nsh, ndh):
    cid = lax.axis_index("c")
    sid = lax.axis_index("s")
    wid = sid * NC + cid

    pltpu.sync_copy(src_hbm.at[pl.ds(wid * CH, CH)], src_v)
    pltpu.sync_copy(dst_hbm.at[pl.ds(wid * CH, CH)], dst_v)
    pltpu.sync_copy(ew_hbm.at[pl.ds(wid * CH, CH)], w_v)

    for hist, (deg_hbm, sh) in enumerate(((degs_hbm, nsh), (degd_hbm, ndh))):
        for w in range(NW):
            pltpu.sync_copy(deg_hbm.at[pl.ds(w * NPAD + sid * NSL, NSL)],
                            part_v.at[w])

        def csum(q, _):
            sl = pl.ds(q * 16, 16)
            acc = part_v[0, sl]
            for w in range(1, NW):
                acc = acc + part_v[w, sl]
            slice_v[sl] = _rsqrt16(acc)
            return 0

        lax.fori_loop(0, NSL // 16, csum, 0)
        pltpu.sync_copy(slice_v, sh.at[pl.ds(sid * NSL, NSL)])

    plsc.subcore_barrier()
    pltpu.sync_copy(nsh, ns_v)
    pltpu.sync_copy(ndh, nd_v)

    def wbody(i, _):
        c = i // 8
        j = (i % 8) * 16
        si = src_v[c, pl.ds(j, 16)]
        di = dst_v[c, pl.ds(j, 16)]
        a = plsc.load_gather(ns_v, [si])
        b = plsc.load_gather(nd_v, [di])
        w_v[c, pl.ds(j, 16)] = w_v[c, pl.ds(j, 16)] * a * b
        return 0

    lax.fori_loop(0, CH * 8, wbody, 0)
    pltpu.sync_copy(w_v, out_hbm.at[pl.ds(wid * CH, CH)])


def _mp_body(g_hbm, src_hbm, dst_hbm, w_hbm, out_hbm,
             src_v, dst_v, w_v, rows_a, rows_b, agg_sh,
             gsem_a, gsem_b, ssem_a, ssem_b):
    cid = lax.axis_index("c")
    sid = lax.axis_index("s")
    wid = sid * NC + cid

    def zfill(r, _):
        for j in range(8):
            rows_a[r, pl.ds(j * 16, 16)] = jnp.zeros((16,), jnp.float32)
        return 0

    lax.fori_loop(0, K, zfill, 0)
    for q in range(RPW // K):
        pltpu.sync_copy(rows_a, agg_sh.at[pl.ds(sid * RPW + q * K, K)])
    pltpu.sync_copy(rows_a.at[pl.ds(0, RPW - (RPW // K) * K)],
                    agg_sh.at[pl.ds(sid * RPW + (RPW // K) * K,
                                    RPW - (RPW // K) * K)])

    @pl.when(sid == NS - 1)
    def _():
        pltpu.sync_copy(rows_a.at[pl.ds(0, N - NS * RPW)],
                        agg_sh.at[pl.ds(NS * RPW, N - NS * RPW)])

    plsc.subcore_barrier()

    def _scale(rows, w_row, c):
        def scale(q, _):
            wv = w_row[c, pl.ds(q * 16, 16)]
            for t in range(16):
                e = q * 16 + t
                bw = jnp.full((16,), wv[t], jnp.float32)
                for j in range(8):
                    sl = pl.ds(j * 16, 16)
                    rows[e, sl] = rows[e, sl] * bw
            return 0

        lax.fori_loop(0, K // 16, scale, 0)

    for half in range(2):
        base = wid * CH + half * HCH
        pltpu.sync_copy(src_hbm.at[pl.ds(base, HCH)], src_v)
        pltpu.sync_copy(dst_hbm.at[pl.ds(base, HCH)], dst_v)
        pltpu.sync_copy(w_hbm.at[pl.ds(base, HCH)], w_v)

        pltpu.async_copy(g_hbm.at[src_v.at[0]], rows_a, gsem_a)

        def pair(p, _):
            c0 = 2 * p
            c1 = 2 * p + 1
            for (c, nxt, rows, gsem, ngsem, ssem, nssem) in (
                    (c0, c1, rows_a, gsem_a, gsem_b, ssem_a, ssem_b),
                    (c1, c0 + 2, rows_b, gsem_b, gsem_a, ssem_b, ssem_a)):
                nrows = rows_b if rows is rows_a else rows_a

                @pl.when(nxt > 1)
                def _():
                    pltpu.make_async_copy(
                        nrows, agg_sh.at[dst_v.at[nxt - 2]], nssem).wait()

                @pl.when(nxt < HCH)
                def _():
                    pltpu.async_copy(g_hbm.at[src_v.at[nxt]], nrows, ngsem)

                pltpu.make_async_copy(g_hbm.at[src_v.at[c]], rows, gsem).wait()
                _scale(rows, w_v, c)
                pltpu.async_copy(rows, agg_sh.at[dst_v.at[c]], ssem, add=True)
            return 0

        lax.fori_loop(0, HCH // 2, pair, 0)

        pltpu.make_async_copy(rows_b, agg_sh.at[dst_v.at[HCH - 1]],
                              ssem_b).wait()

    plsc.subcore_barrier()
    pltpu.sync_copy(agg_sh.at[pl.ds(sid * RPW, RPW)],
                    out_hbm.at[cid, pl.ds(sid * RPW, RPW)])

    @pl.when(sid == NS - 1)
    def _():
        pltpu.sync_copy(agg_sh.at[pl.ds(NS * RPW, N - NS * RPW)],
                        out_hbm.at[cid, pl.ds(NS * RPW, N - NS * RPW)])


def _mm_body(x_ref, w_ref, o_ref):
    o_ref[...] = jnp.dot(x_ref[...], w_ref[...],
                         preferred_element_type=jnp.float32)


def _mid_body(ap_ref, b_ref, w_ref, o_ref):
    h = jnp.maximum(ap_ref[0] + ap_ref[1] + b_ref[0:1, :], 0.0)
    o_ref[...] = jnp.dot(h, w_ref[...], preferred_element_type=jnp.float32)


def _ro_body(ap_ref, nt_ref, b1_ref, w1_ref, bm1_ref, w2_ref, bm2_ref,
             o_ref, cnt_s, acc_v):
    p = pl.program_id(0)
    i = pl.program_id(1)

    @pl.when(jnp.logical_and(p == 0, i == 0))
    def _():
        cnt_s[0] = 0.0

    @pl.when(p == 0)
    def _():
        nt = nt_ref[...]
        col = lax.broadcasted_iota(jnp.int32, nt.shape, 1)
        tgt = jnp.where(col == 2, 1, 0)
        m = jnp.all(nt == tgt, axis=1)
        cnt_s[0] += jnp.sum(m.astype(jnp.float32))

    @pl.when(p == 1)
    def _():
        @pl.when(i == 0)
        def _():
            acc_v[...] = jnp.zeros_like(acc_v)

        dn = cnt_s[0]
        h2 = jnp.maximum(ap_ref[0] + ap_ref[1] + b1_ref[0:1, :], 0.0)
        ridx = (1000 * i +
                lax.broadcasted_iota(jnp.int32, (1000, 1), 0)
                ).astype(jnp.float32)
        msk = (ridx >= (jnp.float32(N) - dn)).astype(jnp.float32)
        acc_v[0:1, :] += jnp.sum(h2 * msk, axis=0, keepdims=True)

        @pl.when(i == 9)
        def _():
            hs = acc_v[0:1, :] / dn
            t1 = jnp.maximum(
                jnp.dot(hs, w1_ref[...],
                        preferred_element_type=jnp.float32) + bm1_ref[0:1, :],
                0.0)
            o = jnp.dot(t1, w2_ref[...],
                        preferred_element_type=jnp.float32) + bm2_ref[0:1, :]
            o_ref[...] = jnp.broadcast_to(o, (8, 128))


def _matmul(x, w):
    return pl.pallas_call(
        _mm_body,
        grid=(N // 1000,),
        in_specs=[
            pl.BlockSpec((1000, F), lambda i: (i, 0)),
            pl.BlockSpec((F, F), lambda i: (0, 0)),
        ],
        out_specs=pl.BlockSpec((1000, F), lambda i: (i, 0)),
        out_shape=jax.ShapeDtypeStruct((N, F), jnp.float32),
    )(x, w)


def _mid(aggp, bb, w):
    return pl.pallas_call(
        _mid_body,
        grid=(N // 1000,),
        in_specs=[
            pl.BlockSpec((2, 1000, F), lambda i: (0, i, 0)),
            pl.BlockSpec((8, F), lambda i: (0, 0)),
            pl.BlockSpec((F, F), lambda i: (0, 0)),
        ],
        out_specs=pl.BlockSpec((1000, F), lambda i: (i, 0)),
        out_shape=jax.ShapeDtypeStruct((N, F), jnp.float32),
    )(aggp, bb, w)


def _readout(aggp, ntp, b1b, w1p, bm1p, w2p, bm2p):
    return pl.pallas_call(
        _ro_body,
        grid=(2, N // 1000),
        in_specs=[
            pl.BlockSpec((2, 1000, F), lambda p, i: (0, i, 0)),
            pl.BlockSpec((1000, F), lambda p, i: (i, 0)),
            pl.BlockSpec((8, F), lambda p, i: (0, 0)),
            pl.BlockSpec((F, F), lambda p, i: (0, 0)),
            pl.BlockSpec((8, F), lambda p, i: (0, 0)),
            pl.BlockSpec((F, F), lambda p, i: (0, 0)),
            pl.BlockSpec((8, F), lambda p, i: (0, 0)),
        ],
        out_specs=pl.BlockSpec((8, F), lambda p, i: (0, 0)),
        out_shape=jax.ShapeDtypeStruct((8, F), jnp.float32),
        scratch_shapes=[
            pltpu.SMEM((1,), jnp.float32),
            pltpu.VMEM((8, F), jnp.float32),
        ],
    )(aggp, ntp, b1b, w1p, bm1p, w2p, bm2p)


def kernel(x, edge_index, edge_attr, node_type, W0, b0, W1, b1,
           mlp_W1, mlp_b1, mlp_W2, mlp_b2):
    src = edge_index[0]
    dst = edge_index[1]

    npad = EPAD - E
    spread = (jnp.arange(npad, dtype=jnp.int32) * 37) % N
    src_p = jnp.concatenate([src, spread]).reshape(NW * CH, K)
    dst_p = jnp.concatenate([dst, (spread * 3 + 11) % N]).reshape(NW * CH, K)
    ew_p = jnp.concatenate(
        [edge_attr, jnp.zeros((npad,), jnp.float32)]).reshape(NW * CH, K)

    _deg_kernel, _w_kernel, _mp_kernel = _sc_kernels()
    degs, degd = _deg_kernel(src, dst)
    xw0 = _matmul(x, W0)
    w_p = _w_kernel(ew_p, src_p, dst_p, degs, degd)

    aggp1 = _mp_kernel(xw0, src_p, dst_p, w_p)

    b0b = jnp.broadcast_to(b0[None, :], (8, F))
    g2 = _mid(aggp1, b0b, W1)
    aggp2 = _mp_kernel(g2, src_p, dst_p, w_p)

    ntp = jnp.pad(node_type, ((0, 0), (0, F - node_type.shape[1])))
    b1b = jnp.broadcast_to(b1[None, :], (8, F))
    w1p = jnp.pad(mlp_W1, ((0, 0), (0, F - mlp_W1.shape[1])))
    bm1p = jnp.broadcast_to(jnp.pad(mlp_b1, (0, F - mlp_b1.shape[0]))[None, :],
                            (8, F))
    w2p = jnp.pad(mlp_W2, ((0, F - mlp_W2.shape[0]), (0, F - mlp_W2.shape[1])))
    bm2p = jnp.broadcast_to(jnp.pad(mlp_b2, (0, F - mlp_b2.shape[0]))[None, :],
                            (8, F))
    out_full = _readout(aggp2, ntp, b1b, w1p, bm1p, w2p, bm2p)
    return out_full[0, :10]


# speedup vs baseline: 1.1538x; 1.1538x over previous
import functools

import jax
import jax.numpy as jnp
from jax import lax
from jax.experimental import pallas as pl
from jax.experimental.pallas import tpu as pltpu
from jax.experimental.pallas import tpu_sc as plsc

N = 10000
E = 320000
F = 128
NC = 2
NS = 16
NW = NC * NS
NPAD = 10240
EPW = E // NW
K = 128
CH = 84
GRP = 12
EPAD = NW * CH * K
RPW = 624

@functools.cache
def _sc_kernels():
    mesh = plsc.VectorSubcoreMesh(core_axis_name="c", subcore_axis_name="s",
                                  num_cores=NC, num_subcores=NS)
    params = pltpu.CompilerParams(needs_layout_passes=False)
    mp_params = pltpu.CompilerParams(needs_layout_passes=False,
                                     internal_scratch_in_bytes=0)
    deg = functools.partial(
        pl.kernel,
        out_type=(jax.ShapeDtypeStruct((NW * NPAD,), jnp.float32),
                  jax.ShapeDtypeStruct((NW * NPAD,), jnp.float32)),
        mesh=mesh,
        compiler_params=params,
        scratch_types=[
            pltpu.VMEM((EPW,), jnp.int32),
            pltpu.VMEM((NPAD,), jnp.float32),
            pltpu.VMEM((NPAD,), jnp.float32),
        ],
    )(_deg_body)
    wk = functools.partial(
        pl.kernel,
        out_type=jax.ShapeDtypeStruct((EPAD,), jnp.float32),
        mesh=mesh,
        compiler_params=params,
        scratch_types=[
            pltpu.VMEM((CH * K,), jnp.int32),
            pltpu.VMEM((CH * K,), jnp.int32),
            pltpu.VMEM((CH * K,), jnp.float32),
            pltpu.VMEM((NPAD,), jnp.float32),
            pltpu.VMEM((NPAD,), jnp.float32),
        ],
    )(_w_body)
    mp = functools.partial(
        pl.kernel,
        out_type=jax.ShapeDtypeStruct((NC, N, F), jnp.float32),
        mesh=mesh,
        compiler_params=mp_params,
        scratch_types=[
            pltpu.VMEM((3, K), jnp.int32),
            pltpu.VMEM((4, K), jnp.int32),
            pltpu.VMEM((3, K), jnp.float32),
            pltpu.VMEM((3 * K, F), jnp.float32),
            pltpu.VMEM_SHARED((N, F), jnp.float32),
            [pltpu.SemaphoreType.DMA] * 4,
            [pltpu.SemaphoreType.DMA] * 3,
            [pltpu.SemaphoreType.DMA] * 3,
        ],
    )(_mp_body)
    return deg, wk, mp


def _deg_body(src_hbm, dst_hbm, outs_hbm, outd_hbm, idx_v, hist_s, hist_d):
    cid = lax.axis_index("c")
    sid = lax.axis_index("s")
    wid = sid * NC + cid

    def zbody(i, _):
        z = jnp.zeros((16,), jnp.float32)
        hist_s[pl.ds(i * 16, 16)] = z
        hist_d[pl.ds(i * 16, 16)] = z
        return 0

    lax.fori_loop(0, NPAD // 16, zbody, 0)

    ones = jnp.ones((16,), jnp.float32)

    pltpu.sync_copy(src_hbm.at[pl.ds(wid * EPW, EPW)], idx_v)

    def sbody(i, _):
        idx = idx_v[pl.ds(i * 16, 16)]
        plsc.addupdate_scatter(hist_s, [idx], ones)
        return 0

    lax.fori_loop(0, EPW // 16, sbody, 0)

    pltpu.sync_copy(dst_hbm.at[pl.ds(wid * EPW, EPW)], idx_v)

    def dbody(i, _):
        idx = idx_v[pl.ds(i * 16, 16)]
        plsc.addupdate_scatter(hist_d, [idx], ones)
        return 0

    lax.fori_loop(0, EPW // 16, dbody, 0)

    pltpu.sync_copy(hist_s, outs_hbm.at[pl.ds(wid * NPAD, NPAD)])
    pltpu.sync_copy(hist_d, outd_hbm.at[pl.ds(wid * NPAD, NPAD)])


def _w_body(ew_hbm, src_hbm, dst_hbm, ns_hbm, nd_hbm, out_hbm,
            src_v, dst_v, w_v, ns_v, nd_v):
    cid = lax.axis_index("c")
    sid = lax.axis_index("s")
    wid = sid * NC + cid
    epw = CH * K

    pltpu.sync_copy(src_hbm.at[pl.ds(wid * epw, epw)], src_v)
    pltpu.sync_copy(dst_hbm.at[pl.ds(wid * epw, epw)], dst_v)
    pltpu.sync_copy(ew_hbm.at[pl.ds(wid * epw, epw)], w_v)
    pltpu.sync_copy(ns_hbm, ns_v)
    pltpu.sync_copy(nd_hbm, nd_v)

    def wbody(i, _):
        sl = pl.ds(i * 16, 16)
        a = plsc.load_gather(ns_v, [src_v[sl]])
        b = plsc.load_gather(nd_v, [dst_v[sl]])
        w_v[sl] = w_v[sl] * a * b
        return 0

    lax.fori_loop(0, epw // 16, wbody, 0)
    pltpu.sync_copy(w_v, out_hbm.at[pl.ds(wid * epw, epw)])


def _mp_body(g_hbm, src_hbm, dst_hbm, w_hbm, out_hbm,
             sr_v, dr_v, wr_v, rowsb, agg_sh, esem, gsem, ssem):
    cid = lax.axis_index("c")
    sid = lax.axis_index("s")
    wid = sid * NC + cid
    rows = tuple(rowsb.at[pl.ds(b * K, K)] for b in range(3))

    def zfill(r, _):
        for j in range(8):
            rowsb[r, pl.ds(j * 16, 16)] = jnp.zeros((16,), jnp.float32)
        return 0

    lax.fori_loop(0, K, zfill, 0)
    for q in range(RPW // K):
        pltpu.sync_copy(rows[0], agg_sh.at[pl.ds(sid * RPW + q * K, K)])
    pltpu.sync_copy(rowsb.at[pl.ds(0, RPW - (RPW // K) * K)],
                    agg_sh.at[pl.ds(sid * RPW + (RPW // K) * K,
                                    RPW - (RPW // K) * K)])

    @pl.when(sid == NS - 1)
    def _():
        pltpu.sync_copy(rowsb.at[pl.ds(0, N - NS * RPW)],
                        agg_sh.at[pl.ds(NS * RPW, N - NS * RPW)])

    plsc.subcore_barrier()

    def _prefetch(c, c3, c4):
        r = wid * CH + c
        pltpu.async_copy(src_hbm.at[r], sr_v.at[pl.ds(c3, 1)], esem[c4])
        pltpu.async_copy(dst_hbm.at[r], dr_v.at[pl.ds(c4, 1)], esem[c4])
        pltpu.async_copy(w_hbm.at[r], wr_v.at[pl.ds(c3, 1)], esem[c4])

    def _ewait(c3, c4):
        pltpu.make_async_copy(src_hbm.at[wid * CH], sr_v.at[pl.ds(c3, 1)],
                              esem[c4]).wait()
        pltpu.make_async_copy(dst_hbm.at[wid * CH], dr_v.at[pl.ds(c4, 1)],
                              esem[c4]).wait()
        pltpu.make_async_copy(w_hbm.at[wid * CH], wr_v.at[pl.ds(c3, 1)],
                              esem[c4]).wait()

    def _scale(c3, b):
        def scale(q, _):
            wv = wr_v[c3, pl.ds(q * 16, 16)]
            for t in range(16):
                e = b * K + q * 16 + t
                bw = jnp.full((16,), wv[t], jnp.float32)
                for j in range(8):
                    sl = pl.ds(j * 16, 16)
                    rowsb[e, sl] = rowsb[e, sl] * bw
            return 0

        lax.fori_loop(0, K // 16, scale, 0)

    def _step(c, c3, c4):
        @pl.when(c >= 2)
        def _():
            pltpu.make_async_copy(
                rows[(c3 + 1) % 3], agg_sh.at[dr_v.at[(c4 + 2) % 4]],
                ssem[(c3 + 1) % 3]).wait()

        @pl.when(c + 1 < CH)
        def _():
            _ewait((c3 + 1) % 3, (c4 + 1) % 4)
            pltpu.async_copy(g_hbm.at[sr_v.at[(c3 + 1) % 3]],
                             rows[(c3 + 1) % 3], gsem[(c3 + 1) % 3])

        @pl.when(c + 2 < CH)
        def _():
            _prefetch(c + 2, (c3 + 2) % 3, (c4 + 2) % 4)

        pltpu.make_async_copy(g_hbm.at[sr_v.at[c3]], rows[c3],
                              gsem[c3]).wait()
        _scale(c3, c3)
        pltpu.async_copy(rows[c3], agg_sh.at[dr_v.at[c4]], ssem[c3],
                         add=True)

    _prefetch(0, 0, 0)
    _prefetch(1, 1, 1)
    _ewait(0, 0)
    pltpu.async_copy(g_hbm.at[sr_v.at[0]], rows[0], gsem[0])

    def group(g, _):
        c_base = g * GRP
        for c0 in range(GRP):
            _step(c_base + c0, c0 % 3, c0 % 4)
        return 0

    lax.fori_loop(0, CH // GRP, group, 0)

    pltpu.make_async_copy(rows[(CH - 2) % 3],
                          agg_sh.at[dr_v.at[(CH - 2) % 4]],
                          ssem[(CH - 2) % 3]).wait()
    pltpu.make_async_copy(rows[(CH - 1) % 3],
                          agg_sh.at[dr_v.at[(CH - 1) % 4]],
                          ssem[(CH - 1) % 3]).wait()

    plsc.subcore_barrier()
    pltpu.sync_copy(agg_sh.at[pl.ds(sid * RPW, RPW)],
                    out_hbm.at[cid, pl.ds(sid * RPW, RPW)])

    @pl.when(sid == NS - 1)
    def _():
        pltpu.sync_copy(agg_sh.at[pl.ds(NS * RPW, N - NS * RPW)],
                        out_hbm.at[cid, pl.ds(NS * RPW, N - NS * RPW)])


def _mm_body(x_ref, w_ref, o_ref):
    o_ref[...] = jnp.dot(x_ref[...], w_ref[...],
                         preferred_element_type=jnp.float32)


def _norm_body(ds_ref, dd_ref, o_ref):
    s = jnp.sum(ds_ref[...], axis=0)
    d = jnp.sum(dd_ref[...], axis=0)
    sd = jnp.stack([s, d], axis=0)
    o_ref[...] = jnp.where(sd > 0.0, lax.rsqrt(sd), 0.0)


def _mid_body(ap_ref, b_ref, w_ref, o_ref):
    h = jnp.maximum(ap_ref[0] + ap_ref[1] + b_ref[0:1, :], 0.0)
    o_ref[...] = jnp.dot(h, w_ref[...], preferred_element_type=jnp.float32)


def _ro_body(ap_ref, nt_ref, b1_ref, w1_ref, bm1_ref, w2_ref, bm2_ref,
             o_ref, cnt_s, acc_v):
    p = pl.program_id(0)
    i = pl.program_id(1)

    @pl.when(jnp.logical_and(p == 0, i == 0))
    def _():
        cnt_s[0] = 0.0

    @pl.when(p == 0)
    def _():
        nt = nt_ref[...]
        col = lax.broadcasted_iota(jnp.int32, nt.shape, 1)
        tgt = jnp.where(col == 2, 1, 0)
        m = jnp.all(nt == tgt, axis=1)
        cnt_s[0] += jnp.sum(m.astype(jnp.float32))

    @pl.when(p == 1)
    def _():
        @pl.when(i == 0)
        def _():
            acc_v[...] = jnp.zeros_like(acc_v)

        dn = cnt_s[0]
        h2 = jnp.maximum(ap_ref[0] + ap_ref[1] + b1_ref[0:1, :], 0.0)
        ridx = (1000 * i +
                lax.broadcasted_iota(jnp.int32, (1000, 1), 0)
                ).astype(jnp.float32)
        msk = (ridx >= (jnp.float32(N) - dn)).astype(jnp.float32)
        acc_v[0:1, :] += jnp.sum(h2 * msk, axis=0, keepdims=True)

        @pl.when(i == 9)
        def _():
            hs = acc_v[0:1, :] / dn
            t1 = jnp.maximum(
                jnp.dot(hs, w1_ref[...],
                        preferred_element_type=jnp.float32) + bm1_ref[0:1, :],
                0.0)
            o = jnp.dot(t1, w2_ref[...],
                        preferred_element_type=jnp.float32) + bm2_ref[0:1, :]
            o_ref[...] = jnp.broadcast_to(o, (8, 128))


def _matmul(x, w):
    return pl.pallas_call(
        _mm_body,
        grid=(N // 1000,),
        in_specs=[
            pl.BlockSpec((1000, F), lambda i: (i, 0)),
            pl.BlockSpec((F, F), lambda i: (0, 0)),
        ],
        out_specs=pl.BlockSpec((1000, F), lambda i: (i, 0)),
        out_shape=jax.ShapeDtypeStruct((N, F), jnp.float32),
    )(x, w)


def _norms(degs, degd):
    return pl.pallas_call(
        _norm_body,
        in_specs=[pl.BlockSpec((NW, NPAD // F, F), lambda: (0, 0, 0)),
                  pl.BlockSpec((NW, NPAD // F, F), lambda: (0, 0, 0))],
        out_specs=pl.BlockSpec((2, NPAD // F, F), lambda: (0, 0, 0)),
        out_shape=jax.ShapeDtypeStruct((2, NPAD // F, F), jnp.float32),
    )(degs.reshape(NW, NPAD // F, F), degd.reshape(NW, NPAD // F, F))


def _mid(aggp, bb, w):
    return pl.pallas_call(
        _mid_body,
        grid=(N // 1000,),
        in_specs=[
            pl.BlockSpec((2, 1000, F), lambda i: (0, i, 0)),
            pl.BlockSpec((8, F), lambda i: (0, 0)),
            pl.BlockSpec((F, F), lambda i: (0, 0)),
        ],
        out_specs=pl.BlockSpec((1000, F), lambda i: (i, 0)),
        out_shape=jax.ShapeDtypeStruct((N, F), jnp.float32),
    )(aggp, bb, w)


def _readout(aggp, ntp, b1b, w1p, bm1p, w2p, bm2p):
    return pl.pallas_call(
        _ro_body,
        grid=(2, N // 1000),
        in_specs=[
            pl.BlockSpec((2, 1000, F), lambda p, i: (0, i, 0)),
            pl.BlockSpec((1000, F), lambda p, i: (i, 0)),
            pl.BlockSpec((8, F), lambda p, i: (0, 0)),
            pl.BlockSpec((F, F), lambda p, i: (0, 0)),
            pl.BlockSpec((8, F), lambda p, i: (0, 0)),
            pl.BlockSpec((F, F), lambda p, i: (0, 0)),
            pl.BlockSpec((8, F), lambda p, i: (0, 0)),
        ],
        out_specs=pl.BlockSpec((8, F), lambda p, i: (0, 0)),
        out_shape=jax.ShapeDtypeStruct((8, F), jnp.float32),
        scratch_shapes=[
            pltpu.SMEM((1,), jnp.float32),
            pltpu.VMEM((8, F), jnp.float32),
        ],
    )(aggp, ntp, b1b, w1p, bm1p, w2p, bm2p)


def kernel(x, edge_index, edge_attr, node_type, W0, b0, W1, b1,
           mlp_W1, mlp_b1, mlp_W2, mlp_b2):
    src = edge_index[0]
    dst = edge_index[1]

    npad = EPAD - E
    spread = (jnp.arange(npad, dtype=jnp.int32) * 37) % N
    src_p = jnp.concatenate([src, spread])
    dst_p = jnp.concatenate([dst, (spread * 3 + 11) % N])
    ew_p = jnp.concatenate([edge_attr, jnp.zeros((npad,), jnp.float32)])

    _deg_kernel, _w_kernel, _mp_kernel = _sc_kernels()
    degs, degd = _deg_kernel(src, dst)
    xw0 = _matmul(x, W0)
    norms = _norms(degs, degd).reshape(2, NPAD)
    w_p = _w_kernel(ew_p, src_p, dst_p, norms[0], norms[1])

    src3 = src_p.reshape(NW * CH, 1, K)
    dst3 = dst_p.reshape(NW * CH, 1, K)
    w3 = w_p.reshape(NW * CH, 1, K)

    aggp1 = _mp_kernel(xw0, src3, dst3, w3)

    b0b = jnp.broadcast_to(b0[None, :], (8, F))
    g2 = _mid(aggp1, b0b, W1)
    aggp2 = _mp_kernel(g2, src3, dst3, w3)

    ntp = jnp.pad(node_type, ((0, 0), (0, F - node_type.shape[1])))
    b1b = jnp.broadcast_to(b1[None, :], (8, F))
    w1p = jnp.pad(mlp_W1, ((0, 0), (0, F - mlp_W1.shape[1])))
    bm1p = jnp.broadcast_to(jnp.pad(mlp_b1, (0, F - mlp_b1.shape[0]))[None, :],
                            (8, F))
    w2p = jnp.pad(mlp_W2, ((0, F - mlp_W2.shape[0]), (0, F - mlp_W2.shape[1])))
    bm2p = jnp.broadcast_to(jnp.pad(mlp_b2, (0, F - mlp_b2.shape[0]))[None, :],
                            (8, F))
    out_full = _readout(aggp2, ntp, b1b, w1p, bm1p, w2p, bm2p)
    return out_full[0, :10]
